# baseline (speedup 1.0000x reference)
import functools

import jax
import jax.numpy as jnp
from jax import lax
from jax.experimental import pallas as pl
from jax.experimental.pallas import tpu as pltpu
from jax.experimental.pallas import tpu_sc as plsc

PAD = 1
EPS = 1e-5

B, S, H = 4, 4096, 1024
NW = 32
TOK_W = (B * S) // NW
SEG_PER_ROW = S // TOK_W
CHUNK = 32
NCHUNK = TOK_W // CHUNK
HC = H // 16


def _vfull(val, dtype=jnp.int32):
    return jnp.full((16,), val, dtype)


def _pad_mask(chunk):
    return jnp.minimum(jnp.abs(chunk - _vfull(PAD)), _vfull(1))


def _sc_body(ids_hbm, word_hbm, pos_hbm, ty_hbm, g_hbm, b_hbm, out_hbm,
             idsbuf, widx, pidx, wrows, prows, tybuf, gbuf, bbuf,
             wsem, psem):
    wid = lax.axis_index("c") * 16 + lax.axis_index("s")
    row = wid // SEG_PER_ROW
    seg = wid % SEG_PER_ROW

    pltpu.sync_copy(ty_hbm, tybuf)
    pltpu.sync_copy(g_hbm, gbuf)
    pltpu.sync_copy(b_hbm, bbuf)

    pltpu.sync_copy(ids_hbm.at[pl.ds(row * (S // 16), S // 16)], idsbuf)

    def pref_body(j, acc):
        return acc + _pad_mask(idsbuf[j])

    accv = lax.fori_loop(0, seg * (TOK_W // 16), pref_body,
                         jnp.zeros((16,), jnp.int32))
    prefv = jnp.broadcast_to(jnp.sum(accv), (16,))

    segbase = seg * (TOK_W // 16)
    for c in range(TOK_W // 16):
        chunk = idsbuf[segbase + c]
        m = _pad_mask(chunk)
        incl = plsc.cumsum(m)
        pid = (prefv + incl) * m + _vfull(PAD)
        cc = c // 2
        k = c % 2
        widx[cc, pl.ds(k * 16, 16)] = chunk
        pidx[cc, pl.ds(k * 16, 16)] = pid
        prefv = prefv + jnp.broadcast_to(jnp.sum(m), (16,))

    base = wid * TOK_W
    inv_h = jnp.float32(1.0 / H)

    def chunk_body(cc, _):
        wd = pltpu.async_copy(word_hbm.at[widx.at[cc]], wrows, wsem)
        pd = pltpu.async_copy(pos_hbm.at[pidx.at[cc]], prows, psem)
        wd.wait()
        pd.wait()

        def tok_body(t, _):
            def p1(j, carry):
                s, q = carry
                v = (wrows[t, pl.ds(j * 16, 16)]
                     + prows[t, pl.ds(j * 16, 16)]
                     + tybuf[j])
                wrows[t, pl.ds(j * 16, 16)] = v
                return s + v, q + v * v

            z = jnp.zeros((16,), jnp.float32)
            s, q = lax.fori_loop(0, HC, p1, (z, z))
            mean = jnp.sum(s) * inv_h
            var = jnp.sum(q) * inv_h - mean * mean
            x = jnp.broadcast_to(var + jnp.float32(EPS), (16,))
            i = plsc.bitcast(x, jnp.int32)
            i = _vfull(0x5F3759DF) - lax.shift_right_logical(i, _vfull(1))
            y = plsc.bitcast(i, jnp.float32)
            c15 = _vfull(1.5, jnp.float32)
            c05 = _vfull(0.5, jnp.float32)
            for _unused in range(3):
                y = y * (c15 - c05 * x * y * y)
            mv = jnp.broadcast_to(mean, (16,))

            def p2(j, _):
                v = wrows[t, pl.ds(j * 16, 16)]
                wrows[t, pl.ds(j * 16, 16)] = (v - mv) * y * gbuf[j] + bbuf[j]
                return 0

            lax.fori_loop(0, HC, p2, 0)
            return 0

        lax.fori_loop(0, CHUNK, tok_body, 0)
        pltpu.sync_copy(wrows, out_hbm.at[pl.ds(base + cc * CHUNK, CHUNK)])
        return 0

    lax.fori_loop(0, NCHUNK, chunk_body, 0)


@functools.partial(
    pl.kernel,
    out_type=jax.ShapeDtypeStruct((B * S, H), jnp.float32),
    mesh=plsc.VectorSubcoreMesh(core_axis_name="c", subcore_axis_name="s"),
    compiler_params=pltpu.CompilerParams(needs_layout_passes=False),
    scratch_types=[
        pltpu.VMEM((S // 16, 16), jnp.int32),
        pltpu.VMEM((NCHUNK, CHUNK), jnp.int32),
        pltpu.VMEM((NCHUNK, CHUNK), jnp.int32),
        pltpu.VMEM((CHUNK, H), jnp.float32),
        pltpu.VMEM((CHUNK, H), jnp.float32),
        pltpu.VMEM((HC, 16), jnp.float32),
        pltpu.VMEM((HC, 16), jnp.float32),
        pltpu.VMEM((HC, 16), jnp.float32),
        pltpu.SemaphoreType.DMA,
        pltpu.SemaphoreType.DMA,
    ],
)
def _sc_embed(ids_hbm, word_hbm, pos_hbm, ty_hbm, g_hbm, b_hbm, out_hbm,
              idsbuf, widx, pidx, wrows, prows, tybuf, gbuf, bbuf,
              wsem, psem):
    _sc_body(ids_hbm, word_hbm, pos_hbm, ty_hbm, g_hbm, b_hbm, out_hbm,
             idsbuf, widx, pidx, wrows, prows, tybuf, gbuf, bbuf,
             wsem, psem)


def kernel(input_ids, token_type_ids, word_emb, pos_emb, type_emb,
           ln_gamma, ln_beta):
    del token_type_ids
    ids2d = input_ids.reshape(B * S // 16, 16)
    out = _sc_embed(ids2d, word_emb, pos_emb,
                    type_emb.reshape(HC, 16),
                    ln_gamma.reshape(HC, 16),
                    ln_beta.reshape(HC, 16))
    return out.reshape(B, S, H)
